# Initial kernel scaffold; baseline (speedup 1.0000x reference)
import jax
import jax.numpy as jnp
from jax.experimental import pallas as pl


def kernel(hn, pos, batch, w_score, W1, b1, W2, b2):
    raise NotImplementedError("write your pallas kernel here")


# speedup vs baseline: 2.9745x; 2.9745x over previous
import functools

import jax
import jax.numpy as jnp
from jax import lax
from jax.experimental import pallas as pl
from jax.experimental.pallas import tpu as pltpu
from jax.experimental.pallas import tpu_sc as plsc

N = 10000
D = 512
K = 2500
KNN = 16
EPS = 1e-8

NP = 10112
NS = 10240
KP = 2560
EP = KP * KNN
BIG = 3e15


def _k1_score(hn, w_score):
    def body(h_ref, w_ref, o_ref):
        o_ref[...] = jnp.dot(h_ref[...], w_ref[...],
                             preferred_element_type=jnp.float32)

    f = pl.pallas_call(
        body, grid=(40,),
        in_specs=[pl.BlockSpec((256, D), lambda i: (i, 0)),
                  pl.BlockSpec((D, 1), lambda i: (0, 0))],
        out_specs=pl.BlockSpec((256, 1), lambda i: (i, 0)),
        out_shape=jax.ShapeDtypeStruct((N, 1), jnp.float32))
    return f(hn, w_score)


def _k2_rank(s_col, s_row):
    def body(sc_ref, sr_ref, o_ref):
        i = pl.program_id(0)
        si = sc_ref[...]
        i_glob = i * 128 + lax.broadcasted_iota(jnp.int32, (128, 1), 0)
        sj = sr_ref[...]
        j_glob = lax.broadcasted_iota(jnp.int32, (1, NP), 1)
        gt = (sj > si).astype(jnp.int32)
        tie = ((sj == si) & (j_glob < i_glob)).astype(jnp.int32)
        o_ref[...] = (jnp.sum(gt, axis=1, keepdims=True)
                      + jnp.sum(tie, axis=1, keepdims=True))

    f = pl.pallas_call(
        body, grid=(NP // 128,),
        in_specs=[pl.BlockSpec((128, 1), lambda i: (i, 0)),
                  pl.BlockSpec((1, NP), lambda i: (0, 0))],
        out_specs=pl.BlockSpec((128, 1), lambda i: (i, 0)),
        out_shape=jax.ShapeDtypeStruct((NP, 1), jnp.int32))
    return f(s_col, s_row)


def _k3_scatter(rank3, ivals3):
    mesh = plsc.VectorSubcoreMesh(core_axis_name="c", subcore_axis_name="s")
    nc = mesh.num_cores

    @functools.partial(
        pl.kernel, mesh=mesh,
        out_type=jax.ShapeDtypeStruct((NS,), jnp.int32),
        scratch_types=[pltpu.VMEM((4, 80), jnp.int32),
                       pltpu.VMEM((4, 80), jnp.int32),
                       pltpu.SemaphoreType.DMA],
    )
    def k(rank_hbm, ival_hbm, out_hbm, rk_v, iv_v, sem):
        wid = lax.axis_index("s") * nc + lax.axis_index("c")
        pltpu.sync_copy(rank_hbm.at[wid], rk_v)
        pltpu.sync_copy(ival_hbm.at[wid], iv_v)
        handles = []
        for j in range(4):
            handles.append(
                pltpu.async_copy(iv_v.at[j], out_hbm.at[rk_v.at[j]], sem))
        for h in handles:
            h.wait()

    return k(rank3, ivals3)


def _k4_gather(idx_full, hn, px, py, pz, batch, s1):
    mesh = plsc.VectorSubcoreMesh(core_axis_name="c", subcore_axis_name="s")
    nc = mesh.num_cores

    @functools.partial(
        pl.kernel, mesh=mesh,
        out_type=[jax.ShapeDtypeStruct((KP, D), jnp.float32),
                  jax.ShapeDtypeStruct((KP,), jnp.float32),
                  jax.ShapeDtypeStruct((KP,), jnp.float32),
                  jax.ShapeDtypeStruct((KP,), jnp.float32),
                  jax.ShapeDtypeStruct((KP,), jnp.int32),
                  jax.ShapeDtypeStruct((KP,), jnp.float32)],
        scratch_types=[pltpu.VMEM((80,), jnp.int32),
                       pltpu.VMEM((80, D), jnp.float32),
                       pltpu.VMEM((80,), jnp.float32),
                       pltpu.VMEM((80,), jnp.float32),
                       pltpu.VMEM((80,), jnp.float32),
                       pltpu.VMEM((80,), jnp.int32),
                       pltpu.VMEM((80,), jnp.float32),
                       pltpu.SemaphoreType.DMA],
    )
    def k(idx_hbm, hn_hbm, px_hbm, py_hbm, pz_hbm, b_hbm, s_hbm,
          ghn_o, px_o, py_o, pz_o, b_o, s_o,
          idx_v, rows_v, pxv, pyv, pzv, bv, sv, sem):
        wid = lax.axis_index("s") * nc + lax.axis_index("c")
        base = wid * 80
        pltpu.sync_copy(idx_hbm.at[pl.ds(base, 80)], idx_v)
        hs = [pltpu.async_copy(hn_hbm.at[idx_v], rows_v, sem),
              pltpu.async_copy(px_hbm.at[idx_v], pxv, sem),
              pltpu.async_copy(py_hbm.at[idx_v], pyv, sem),
              pltpu.async_copy(pz_hbm.at[idx_v], pzv, sem),
              pltpu.async_copy(b_hbm.at[idx_v], bv, sem),
              pltpu.async_copy(s_hbm.at[idx_v], sv, sem)]
        for h in hs:
            h.wait()
        pltpu.sync_copy(rows_v, ghn_o.at[pl.ds(base, 80)])
        pltpu.sync_copy(pxv, px_o.at[pl.ds(base, 80)])
        pltpu.sync_copy(pyv, py_o.at[pl.ds(base, 80)])
        pltpu.sync_copy(pzv, pz_o.at[pl.ds(base, 80)])
        pltpu.sync_copy(bv, b_o.at[pl.ds(base, 80)])
        pltpu.sync_copy(sv, s_o.at[pl.ds(base, 80)])

    return k(idx_full, hn, px, py, pz, batch, s1)


def _k5_knn(ghn, ssc_col, px_col, py_col, pz_col, px_row, py_row, pz_row):
    def body(g_ref, s_ref, pxc, pyc, pzc, pxr, pyr, pzr,
             subhn_ref, nx_ref, ny_ref, nz_ref):
        i = pl.program_id(0)
        i_glob = i * 128 + lax.broadcasted_iota(jnp.int32, (128, 1), 0)
        j_glob = lax.broadcasted_iota(jnp.int32, (1, KP), 1)
        xr, yr, zr = pxr[...], pyr[...], pzr[...]
        dx = pxc[...] - xr
        dy = pyc[...] - yr
        dz = pzc[...] - zr
        d2 = dx * dx + dy * dy + dz * dz
        d2 = jnp.where(j_glob == i_glob, 1e9, d2)
        d2 = jnp.where(j_glob >= K, BIG, d2)
        for t in range(KNN):
            m = jnp.min(d2, axis=1, keepdims=True)
            jsel = jnp.min(jnp.where(d2 == m, j_glob, jnp.int32(2 ** 30)),
                           axis=1, keepdims=True)
            selm = j_glob == jsel
            nx_ref[:, t:t + 1] = jnp.sum(jnp.where(selm, xr, 0.0),
                                         axis=1, keepdims=True)
            ny_ref[:, t:t + 1] = jnp.sum(jnp.where(selm, yr, 0.0),
                                         axis=1, keepdims=True)
            nz_ref[:, t:t + 1] = jnp.sum(jnp.where(selm, zr, 0.0),
                                         axis=1, keepdims=True)
            d2 = jnp.where(selm, BIG, d2)
        gate = 1.0 / (1.0 + jnp.exp(-s_ref[...]))
        subhn_ref[...] = g_ref[...] * gate

    f = pl.pallas_call(
        body, grid=(KP // 128,),
        in_specs=[pl.BlockSpec((128, D), lambda i: (i, 0)),
                  pl.BlockSpec((128, 1), lambda i: (i, 0)),
                  pl.BlockSpec((128, 1), lambda i: (i, 0)),
                  pl.BlockSpec((128, 1), lambda i: (i, 0)),
                  pl.BlockSpec((128, 1), lambda i: (i, 0)),
                  pl.BlockSpec((1, KP), lambda i: (0, 0)),
                  pl.BlockSpec((1, KP), lambda i: (0, 0)),
                  pl.BlockSpec((1, KP), lambda i: (0, 0))],
        out_specs=[pl.BlockSpec((128, D), lambda i: (i, 0)),
                   pl.BlockSpec((128, KNN), lambda i: (i, 0)),
                   pl.BlockSpec((128, KNN), lambda i: (i, 0)),
                   pl.BlockSpec((128, KNN), lambda i: (i, 0))],
        out_shape=[jax.ShapeDtypeStruct((KP, D), jnp.float32),
                   jax.ShapeDtypeStruct((KP, KNN), jnp.float32),
                   jax.ShapeDtypeStruct((KP, KNN), jnp.float32),
                   jax.ShapeDtypeStruct((KP, KNN), jnp.float32)])
    return f(ghn, ssc_col, px_col, py_col, pz_col, px_row, py_row, pz_row)


def _k6_edges(nx, ny, nz, sx, sy, sz, W1p, b1, W2, b2):
    def body(nx_ref, ny_ref, nz_ref, sx_ref, sy_ref, sz_ref,
             w1_ref, b1_ref, w2_ref, b2_ref, o_ref):
        dx = nx_ref[...] - sx_ref[...]
        dy = ny_ref[...] - sy_ref[...]
        dz = nz_ref[...] - sz_ref[...]
        norm = jnp.sqrt(dx * dx + dy * dy + dz * dz + EPS)
        x = dx / norm
        y = dy / norm
        z = dz / norm
        zero = jnp.zeros_like(x)
        fe = jnp.concatenate(
            [jnp.ones_like(x), x, y, z,
             x * y, y * z, 3.0 * z * z - 1.0, x * z, x * x - y * y,
             zero, zero, zero, zero, zero, zero, zero], axis=1)
        h1 = jnp.dot(fe, w1_ref[...], preferred_element_type=jnp.float32)
        h1 = jnp.maximum(h1 + b1_ref[...], 0.0)
        he = jnp.dot(h1, w2_ref[...], preferred_element_type=jnp.float32)
        o_ref[...] = he + b2_ref[...]

    f = pl.pallas_call(
        body, grid=(EP // 512,),
        in_specs=[pl.BlockSpec((512, 1), lambda i: (i, 0))] * 6 +
                 [pl.BlockSpec((16, D), lambda i: (0, 0)),
                  pl.BlockSpec((1, D), lambda i: (0, 0)),
                  pl.BlockSpec((D, D), lambda i: (0, 0)),
                  pl.BlockSpec((1, D), lambda i: (0, 0))],
        out_specs=pl.BlockSpec((512, D), lambda i: (i, 0)),
        out_shape=jax.ShapeDtypeStruct((EP, D), jnp.float32))
    return f(nx, ny, nz, sx, sy, sz, W1p, b1, W2, b2)


def kernel(hn, pos, batch, w_score, W1, b1, W2, b2):
    score = _k1_score(hn, w_score)
    s1 = score.reshape(N)
    spad = jnp.concatenate(
        [s1, jnp.full((NP - N,), -jnp.inf, dtype=jnp.float32)])
    rank = _k2_rank(spad.reshape(NP, 1), spad.reshape(1, NP))
    rank_full = jnp.concatenate(
        [rank[:N, 0], N + jnp.arange(NS - N, dtype=jnp.int32)])
    ivals = jnp.arange(NS, dtype=jnp.int32)
    idx_by_rank = _k3_scatter(rank_full.reshape(32, 4, 80),
                              ivals.reshape(32, 4, 80))
    posT = pos.T
    ghn, spx, spy, spz, sbatch, ssc = _k4_gather(
        idx_by_rank, hn, posT[0], posT[1], posT[2], batch, s1)

    subhn_full, pxn, pyn, pzn = _k5_knn(
        ghn, ssc.reshape(KP, 1),
        spx.reshape(KP, 1), spy.reshape(KP, 1), spz.reshape(KP, 1),
        spx.reshape(1, KP), spy.reshape(1, KP), spz.reshape(1, KP))

    W1p = jnp.pad(W1, ((0, 16 - W1.shape[0]), (0, 0)))
    he_full = _k6_edges(
        pxn.reshape(EP, 1), pyn.reshape(EP, 1), pzn.reshape(EP, 1),
        jnp.broadcast_to(spx[:, None], (KP, KNN)).reshape(EP, 1),
        jnp.broadcast_to(spy[:, None], (KP, KNN)).reshape(EP, 1),
        jnp.broadcast_to(spz[:, None], (KP, KNN)).reshape(EP, 1),
        W1p, b1.reshape(1, D), W2, b2.reshape(1, D))

    sub_hn = subhn_full[:K]
    sub_pos = jnp.stack([spx[:K], spy[:K], spz[:K]], axis=1)
    sub_batch = sbatch[:K]
    he = he_full[:K * KNN]
    return sub_hn, sub_pos, sub_batch, he


# speedup vs baseline: 3.6199x; 1.2170x over previous
import functools

import jax
import jax.numpy as jnp
from jax import lax
from jax.experimental import pallas as pl
from jax.experimental.pallas import tpu as pltpu
from jax.experimental.pallas import tpu_sc as plsc

N = 10000
D = 512
K = 2500
KNN = 16
EPS = 1e-8

NP = 10112
NS = 10240
KP = 2560
EP = KP * KNN
BIG = 3e15


def _k1_score(hn, w_score):
    def body(h_ref, w_ref, o_ref):
        o_ref[...] = jnp.dot(h_ref[...], w_ref[...],
                             preferred_element_type=jnp.float32)

    f = pl.pallas_call(
        body, grid=(40,),
        in_specs=[pl.BlockSpec((256, D), lambda i: (i, 0)),
                  pl.BlockSpec((D, 1), lambda i: (0, 0))],
        out_specs=pl.BlockSpec((256, 1), lambda i: (i, 0)),
        out_shape=jax.ShapeDtypeStruct((N, 1), jnp.float32))
    return f(hn, w_score)


def _k2_rank(s_col, s_row):
    def body(sc_ref, sr_ref, o_ref):
        i = pl.program_id(0)
        si = sc_ref[...]
        i_glob = i * 128 + lax.broadcasted_iota(jnp.int32, (128, 1), 0)
        sj = sr_ref[...]
        j_glob = lax.broadcasted_iota(jnp.int32, (1, NP), 1)
        gt = (sj > si).astype(jnp.int32)
        tie = ((sj == si) & (j_glob < i_glob)).astype(jnp.int32)
        o_ref[...] = (jnp.sum(gt, axis=1, keepdims=True)
                      + jnp.sum(tie, axis=1, keepdims=True))

    f = pl.pallas_call(
        body, grid=(NP // 128,),
        in_specs=[pl.BlockSpec((128, 1), lambda i: (i, 0)),
                  pl.BlockSpec((1, NP), lambda i: (0, 0))],
        out_specs=pl.BlockSpec((128, 1), lambda i: (i, 0)),
        out_shape=jax.ShapeDtypeStruct((NP, 1), jnp.int32))
    return f(s_col, s_row)


def _k3_scatter(rank3, ivals3):
    mesh = plsc.VectorSubcoreMesh(core_axis_name="c", subcore_axis_name="s")
    nc = mesh.num_cores

    @functools.partial(
        pl.kernel, mesh=mesh,
        out_type=jax.ShapeDtypeStruct((NS,), jnp.int32),
        scratch_types=[pltpu.VMEM((4, 80), jnp.int32),
                       pltpu.VMEM((4, 80), jnp.int32),
                       pltpu.SemaphoreType.DMA],
    )
    def k(rank_hbm, ival_hbm, out_hbm, rk_v, iv_v, sem):
        wid = lax.axis_index("s") * nc + lax.axis_index("c")
        pltpu.sync_copy(rank_hbm.at[wid], rk_v)
        pltpu.sync_copy(ival_hbm.at[wid], iv_v)
        handles = []
        for j in range(4):
            handles.append(
                pltpu.async_copy(iv_v.at[j], out_hbm.at[rk_v.at[j]], sem))
        for h in handles:
            h.wait()

    return k(rank3, ivals3)


def _k4_gather(idx_full, hn, px, py, pz, batch, s1):
    mesh = plsc.VectorSubcoreMesh(core_axis_name="c", subcore_axis_name="s")
    nc = mesh.num_cores

    @functools.partial(
        pl.kernel, mesh=mesh,
        out_type=[jax.ShapeDtypeStruct((KP, D), jnp.float32),
                  jax.ShapeDtypeStruct((KP,), jnp.float32),
                  jax.ShapeDtypeStruct((KP,), jnp.float32),
                  jax.ShapeDtypeStruct((KP,), jnp.float32),
                  jax.ShapeDtypeStruct((KP,), jnp.int32),
                  jax.ShapeDtypeStruct((KP,), jnp.float32)],
        scratch_types=[pltpu.VMEM((80,), jnp.int32),
                       pltpu.VMEM((80, D), jnp.float32),
                       pltpu.VMEM((80,), jnp.float32),
                       pltpu.VMEM((80,), jnp.float32),
                       pltpu.VMEM((80,), jnp.float32),
                       pltpu.VMEM((80,), jnp.int32),
                       pltpu.VMEM((80,), jnp.float32),
                       pltpu.SemaphoreType.DMA],
    )
    def k(idx_hbm, hn_hbm, px_hbm, py_hbm, pz_hbm, b_hbm, s_hbm,
          ghn_o, px_o, py_o, pz_o, b_o, s_o,
          idx_v, rows_v, pxv, pyv, pzv, bv, sv, sem):
        wid = lax.axis_index("s") * nc + lax.axis_index("c")
        base = wid * 80
        pltpu.sync_copy(idx_hbm.at[pl.ds(base, 80)], idx_v)
        hs = [pltpu.async_copy(hn_hbm.at[idx_v], rows_v, sem),
              pltpu.async_copy(px_hbm.at[idx_v], pxv, sem),
              pltpu.async_copy(py_hbm.at[idx_v], pyv, sem),
              pltpu.async_copy(pz_hbm.at[idx_v], pzv, sem),
              pltpu.async_copy(b_hbm.at[idx_v], bv, sem),
              pltpu.async_copy(s_hbm.at[idx_v], sv, sem)]
        for h in hs:
            h.wait()
        pltpu.sync_copy(rows_v, ghn_o.at[pl.ds(base, 80)])
        pltpu.sync_copy(pxv, px_o.at[pl.ds(base, 80)])
        pltpu.sync_copy(pyv, py_o.at[pl.ds(base, 80)])
        pltpu.sync_copy(pzv, pz_o.at[pl.ds(base, 80)])
        pltpu.sync_copy(bv, b_o.at[pl.ds(base, 80)])
        pltpu.sync_copy(sv, s_o.at[pl.ds(base, 80)])

    return k(idx_full, hn, px, py, pz, batch, s1)


def _k5_knn(ghn, ssc_col, px_col, py_col, pz_col, px_row, py_row, pz_row):
    def body(g_ref, s_ref, pxc, pyc, pzc, pxr, pyr, pzr,
             subhn_ref, nbr_ref):
        i = pl.program_id(0)
        i_glob = i * 128 + lax.broadcasted_iota(jnp.int32, (128, 1), 0)
        j_glob = lax.broadcasted_iota(jnp.int32, (1, KP), 1)
        dx = pxc[...] - pxr[...]
        dy = pyc[...] - pyr[...]
        dz = pzc[...] - pzr[...]
        d2 = dx * dx + dy * dy + dz * dz
        d2 = jnp.where(j_glob == i_glob, 1e9, d2)
        d2 = jnp.where(j_glob >= K, BIG, d2)
        for t in range(KNN):
            m = jnp.min(d2, axis=1, keepdims=True)
            jsel = jnp.min(jnp.where(d2 == m, j_glob, jnp.int32(2 ** 30)),
                           axis=1, keepdims=True)
            nbr_ref[:, t:t + 1] = jsel
            d2 = jnp.where(j_glob == jsel, BIG, d2)
        gate = 1.0 / (1.0 + jnp.exp(-s_ref[...]))
        subhn_ref[...] = g_ref[...] * gate

    f = pl.pallas_call(
        body, grid=(KP // 128,),
        in_specs=[pl.BlockSpec((128, D), lambda i: (i, 0)),
                  pl.BlockSpec((128, 1), lambda i: (i, 0)),
                  pl.BlockSpec((128, 1), lambda i: (i, 0)),
                  pl.BlockSpec((128, 1), lambda i: (i, 0)),
                  pl.BlockSpec((128, 1), lambda i: (i, 0)),
                  pl.BlockSpec((1, KP), lambda i: (0, 0)),
                  pl.BlockSpec((1, KP), lambda i: (0, 0)),
                  pl.BlockSpec((1, KP), lambda i: (0, 0))],
        out_specs=[pl.BlockSpec((128, D), lambda i: (i, 0)),
                   pl.BlockSpec((128, KNN), lambda i: (i, 0))],
        out_shape=[jax.ShapeDtypeStruct((K, D), jnp.float32),
                   jax.ShapeDtypeStruct((KP, KNN), jnp.int32)])
    return f(ghn, ssc_col, px_col, py_col, pz_col, px_row, py_row, pz_row)


def _k5b_coords(nbr3, spx, spy, spz):
    mesh = plsc.VectorSubcoreMesh(core_axis_name="c", subcore_axis_name="s")
    nc = mesh.num_cores

    @functools.partial(
        pl.kernel, mesh=mesh,
        out_type=[jax.ShapeDtypeStruct((32, 10, 128), jnp.float32),
                  jax.ShapeDtypeStruct((32, 10, 128), jnp.float32),
                  jax.ShapeDtypeStruct((32, 10, 128), jnp.float32)],
        scratch_types=[pltpu.VMEM((10, 128), jnp.int32),
                       pltpu.VMEM((10, 128), jnp.float32),
                       pltpu.VMEM((10, 128), jnp.float32),
                       pltpu.VMEM((10, 128), jnp.float32),
                       pltpu.SemaphoreType.DMA],
    )
    def k(nbr_hbm, px_hbm, py_hbm, pz_hbm, nx_o, ny_o, nz_o,
          idx_v, xv, yv, zv, sem):
        wid = lax.axis_index("s") * nc + lax.axis_index("c")
        pltpu.sync_copy(nbr_hbm.at[wid], idx_v)
        hs = []
        for c in range(10):
            hs.append(pltpu.async_copy(px_hbm.at[idx_v.at[c]], xv.at[c], sem))
            hs.append(pltpu.async_copy(py_hbm.at[idx_v.at[c]], yv.at[c], sem))
            hs.append(pltpu.async_copy(pz_hbm.at[idx_v.at[c]], zv.at[c], sem))
        for h in hs:
            h.wait()
        pltpu.sync_copy(xv, nx_o.at[wid])
        pltpu.sync_copy(yv, ny_o.at[wid])
        pltpu.sync_copy(zv, nz_o.at[wid])

    return k(nbr3, spx, spy, spz)


def _k6_edges(nx, ny, nz, sx, sy, sz, W1p, b1, W2, b2):
    def body(nx_ref, ny_ref, nz_ref, sx_ref, sy_ref, sz_ref,
             w1_ref, b1_ref, w2_ref, b2_ref, o_ref):
        dx = nx_ref[...] - sx_ref[...]
        dy = ny_ref[...] - sy_ref[...]
        dz = nz_ref[...] - sz_ref[...]
        norm = jnp.sqrt(dx * dx + dy * dy + dz * dz + EPS)
        x = dx / norm
        y = dy / norm
        z = dz / norm
        zero = jnp.zeros_like(x)
        fe = jnp.concatenate(
            [jnp.ones_like(x), x, y, z,
             x * y, y * z, 3.0 * z * z - 1.0, x * z, x * x - y * y,
             zero, zero, zero, zero, zero, zero, zero], axis=1)
        h1 = jnp.dot(fe, w1_ref[...], preferred_element_type=jnp.float32)
        h1 = jnp.maximum(h1 + b1_ref[...], 0.0)
        he = jnp.dot(h1, w2_ref[...], preferred_element_type=jnp.float32)
        o_ref[...] = he + b2_ref[...]

    f = pl.pallas_call(
        body, grid=(79,),
        in_specs=[pl.BlockSpec((512, 1), lambda i: (i, 0))] * 6 +
                 [pl.BlockSpec((16, D), lambda i: (0, 0)),
                  pl.BlockSpec((1, D), lambda i: (0, 0)),
                  pl.BlockSpec((D, D), lambda i: (0, 0)),
                  pl.BlockSpec((1, D), lambda i: (0, 0))],
        out_specs=pl.BlockSpec((512, D), lambda i: (i, 0)),
        out_shape=jax.ShapeDtypeStruct((K * KNN, D), jnp.float32))
    return f(nx, ny, nz, sx, sy, sz, W1p, b1, W2, b2)


def kernel(hn, pos, batch, w_score, W1, b1, W2, b2):
    score = _k1_score(hn, w_score)
    s1 = score.reshape(N)
    spad = jnp.concatenate(
        [s1, jnp.full((NP - N,), -jnp.inf, dtype=jnp.float32)])
    rank = _k2_rank(spad.reshape(NP, 1), spad.reshape(1, NP))
    rank_full = jnp.concatenate(
        [rank[:N, 0], N + jnp.arange(NS - N, dtype=jnp.int32)])
    ivals = jnp.arange(NS, dtype=jnp.int32)
    idx_by_rank = _k3_scatter(rank_full.reshape(32, 4, 80),
                              ivals.reshape(32, 4, 80))
    posT = pos.T
    ghn, spx, spy, spz, sbatch, ssc = _k4_gather(
        idx_by_rank, hn, posT[0], posT[1], posT[2], batch, s1)

    sub_hn, nbr = _k5_knn(
        ghn, ssc.reshape(KP, 1),
        spx.reshape(KP, 1), spy.reshape(KP, 1), spz.reshape(KP, 1),
        spx.reshape(1, KP), spy.reshape(1, KP), spz.reshape(1, KP))

    nxg, nyg, nzg = _k5b_coords(nbr.reshape(32, 10, 128), spx, spy, spz)

    W1p = jnp.pad(W1, ((0, 16 - W1.shape[0]), (0, 0)))
    he = _k6_edges(
        nxg.reshape(EP, 1), nyg.reshape(EP, 1), nzg.reshape(EP, 1),
        jnp.broadcast_to(spx[:, None], (KP, KNN)).reshape(EP, 1),
        jnp.broadcast_to(spy[:, None], (KP, KNN)).reshape(EP, 1),
        jnp.broadcast_to(spz[:, None], (KP, KNN)).reshape(EP, 1),
        W1p, b1.reshape(1, D), W2, b2.reshape(1, D))

    sub_pos = jnp.stack([spx[:K], spy[:K], spz[:K]], axis=1)
    sub_batch = sbatch[:K]
    return sub_hn, sub_pos, sub_batch, he


# speedup vs baseline: 3.6837x; 1.0176x over previous
import functools

import jax
import jax.numpy as jnp
from jax import lax
from jax.experimental import pallas as pl
from jax.experimental.pallas import tpu as pltpu
from jax.experimental.pallas import tpu_sc as plsc

N = 10000
D = 512
K = 2500
KNN = 16
EPS = 1e-8

NP = 10112
NS = 10240
KP = 2560
EP = KP * KNN
BIG = 3e15


def _k1_score(hn, w_score):
    def body(h_ref, w_ref, o_ref):
        o_ref[...] = jnp.dot(h_ref[...], w_ref[...],
                             preferred_element_type=jnp.float32)

    f = pl.pallas_call(
        body, grid=(40,),
        in_specs=[pl.BlockSpec((256, D), lambda i: (i, 0)),
                  pl.BlockSpec((D, 1), lambda i: (0, 0))],
        out_specs=pl.BlockSpec((256, 1), lambda i: (i, 0)),
        out_shape=jax.ShapeDtypeStruct((N, 1), jnp.float32))
    return f(hn, w_score)


def _k2_rank(s_col, s_row):
    def body(sc_ref, sr_ref, o_ref):
        i = pl.program_id(0)
        si = sc_ref[...]
        i_glob = i * 128 + lax.broadcasted_iota(jnp.int32, (128, 1), 0)
        sj = sr_ref[...]
        j_glob = lax.broadcasted_iota(jnp.int32, (1, NP), 1)
        beats = (sj > si) | ((sj == si) & (j_glob < i_glob))
        o_ref[...] = jnp.sum(beats.astype(jnp.int32), axis=1, keepdims=True)

    f = pl.pallas_call(
        body, grid=(NP // 128,),
        in_specs=[pl.BlockSpec((128, 1), lambda i: (i, 0)),
                  pl.BlockSpec((1, NP), lambda i: (0, 0))],
        out_specs=pl.BlockSpec((128, 1), lambda i: (i, 0)),
        out_shape=jax.ShapeDtypeStruct((NP, 1), jnp.int32))
    return f(s_col, s_row)


def _k3_scatter(rank3, ivals3):
    mesh = plsc.VectorSubcoreMesh(core_axis_name="c", subcore_axis_name="s")
    nc = mesh.num_cores

    @functools.partial(
        pl.kernel, mesh=mesh,
        out_type=jax.ShapeDtypeStruct((NS,), jnp.int32),
        scratch_types=[pltpu.VMEM((4, 80), jnp.int32),
                       pltpu.VMEM((4, 80), jnp.int32),
                       pltpu.SemaphoreType.DMA],
    )
    def k(rank_hbm, ival_hbm, out_hbm, rk_v, iv_v, sem):
        wid = lax.axis_index("s") * nc + lax.axis_index("c")
        pltpu.sync_copy(rank_hbm.at[wid], rk_v)
        pltpu.sync_copy(ival_hbm.at[wid], iv_v)
        handles = []
        for j in range(4):
            handles.append(
                pltpu.async_copy(iv_v.at[j], out_hbm.at[rk_v.at[j]], sem))
        for h in handles:
            h.wait()

    return k(rank3, ivals3)


def _k4_gather(idx_full, hn, px, py, pz, batch, s1):
    mesh = plsc.VectorSubcoreMesh(core_axis_name="c", subcore_axis_name="s")
    nc = mesh.num_cores

    @functools.partial(
        pl.kernel, mesh=mesh,
        out_type=[jax.ShapeDtypeStruct((KP, D), jnp.float32),
                  jax.ShapeDtypeStruct((KP,), jnp.float32),
                  jax.ShapeDtypeStruct((KP,), jnp.float32),
                  jax.ShapeDtypeStruct((KP,), jnp.float32),
                  jax.ShapeDtypeStruct((KP,), jnp.int32),
                  jax.ShapeDtypeStruct((KP,), jnp.float32)],
        scratch_types=[pltpu.VMEM((80,), jnp.int32),
                       pltpu.VMEM((80, D), jnp.float32),
                       pltpu.VMEM((80,), jnp.float32),
                       pltpu.VMEM((80,), jnp.float32),
                       pltpu.VMEM((80,), jnp.float32),
                       pltpu.VMEM((80,), jnp.int32),
                       pltpu.VMEM((80,), jnp.float32),
                       pltpu.SemaphoreType.DMA],
    )
    def k(idx_hbm, hn_hbm, px_hbm, py_hbm, pz_hbm, b_hbm, s_hbm,
          ghn_o, px_o, py_o, pz_o, b_o, s_o,
          idx_v, rows_v, pxv, pyv, pzv, bv, sv, sem):
        wid = lax.axis_index("s") * nc + lax.axis_index("c")
        base = wid * 80
        pltpu.sync_copy(idx_hbm.at[pl.ds(base, 80)], idx_v)
        hs = [pltpu.async_copy(hn_hbm.at[idx_v], rows_v, sem),
              pltpu.async_copy(px_hbm.at[idx_v], pxv, sem),
              pltpu.async_copy(py_hbm.at[idx_v], pyv, sem),
              pltpu.async_copy(pz_hbm.at[idx_v], pzv, sem),
              pltpu.async_copy(b_hbm.at[idx_v], bv, sem),
              pltpu.async_copy(s_hbm.at[idx_v], sv, sem)]
        for h in hs:
            h.wait()
        pltpu.sync_copy(rows_v, ghn_o.at[pl.ds(base, 80)])
        pltpu.sync_copy(pxv, px_o.at[pl.ds(base, 80)])
        pltpu.sync_copy(pyv, py_o.at[pl.ds(base, 80)])
        pltpu.sync_copy(pzv, pz_o.at[pl.ds(base, 80)])
        pltpu.sync_copy(bv, b_o.at[pl.ds(base, 80)])
        pltpu.sync_copy(sv, s_o.at[pl.ds(base, 80)])

    return k(idx_full, hn, px, py, pz, batch, s1)


def _k5_knn(ghn, ssc_col, px_col, py_col, pz_col, px_row, py_row, pz_row):
    def body(g_ref, s_ref, pxc, pyc, pzc, pxr, pyr, pzr,
             subhn_ref, nbr_ref):
        i = pl.program_id(0)
        i_glob = i * 128 + lax.broadcasted_iota(jnp.int32, (128, 1), 0)
        j_glob = lax.broadcasted_iota(jnp.int32, (1, KP), 1)
        dx = pxc[...] - pxr[...]
        dy = pyc[...] - pyr[...]
        dz = pzc[...] - pzr[...]
        d2 = dx * dx + dy * dy + dz * dz
        d2 = jnp.where(j_glob == i_glob, 1e9, d2)
        d2 = jnp.where(j_glob >= K, BIG, d2)
        for t in range(KNN):
            m = jnp.min(d2, axis=1, keepdims=True)
            jsel = jnp.min(jnp.where(d2 == m, j_glob, jnp.int32(2 ** 30)),
                           axis=1, keepdims=True)
            nbr_ref[:, t:t + 1] = jsel
            d2 = jnp.where(j_glob == jsel, BIG, d2)
        gate = 1.0 / (1.0 + jnp.exp(-s_ref[...]))
        subhn_ref[...] = g_ref[...] * gate

    f = pl.pallas_call(
        body, grid=(KP // 128,),
        in_specs=[pl.BlockSpec((128, D), lambda i: (i, 0)),
                  pl.BlockSpec((128, 1), lambda i: (i, 0)),
                  pl.BlockSpec((128, 1), lambda i: (i, 0)),
                  pl.BlockSpec((128, 1), lambda i: (i, 0)),
                  pl.BlockSpec((128, 1), lambda i: (i, 0)),
                  pl.BlockSpec((1, KP), lambda i: (0, 0)),
                  pl.BlockSpec((1, KP), lambda i: (0, 0)),
                  pl.BlockSpec((1, KP), lambda i: (0, 0))],
        out_specs=[pl.BlockSpec((128, D), lambda i: (i, 0)),
                   pl.BlockSpec((128, KNN), lambda i: (i, 0))],
        out_shape=[jax.ShapeDtypeStruct((K, D), jnp.float32),
                   jax.ShapeDtypeStruct((KP, KNN), jnp.int32)])
    return f(ghn, ssc_col, px_col, py_col, pz_col, px_row, py_row, pz_row)


def _k5b_coords(nbr3, spx, spy, spz):
    mesh = plsc.VectorSubcoreMesh(core_axis_name="c", subcore_axis_name="s")
    nc = mesh.num_cores

    @functools.partial(
        pl.kernel, mesh=mesh,
        out_type=[jax.ShapeDtypeStruct((32, 10, 128), jnp.float32),
                  jax.ShapeDtypeStruct((32, 10, 128), jnp.float32),
                  jax.ShapeDtypeStruct((32, 10, 128), jnp.float32)],
        scratch_types=[pltpu.VMEM((10, 128), jnp.int32),
                       pltpu.VMEM((10, 128), jnp.float32),
                       pltpu.VMEM((10, 128), jnp.float32),
                       pltpu.VMEM((10, 128), jnp.float32),
                       pltpu.SemaphoreType.DMA],
    )
    def k(nbr_hbm, px_hbm, py_hbm, pz_hbm, nx_o, ny_o, nz_o,
          idx_v, xv, yv, zv, sem):
        wid = lax.axis_index("s") * nc + lax.axis_index("c")
        pltpu.sync_copy(nbr_hbm.at[wid], idx_v)
        hs = []
        for c in range(10):
            hs.append(pltpu.async_copy(px_hbm.at[idx_v.at[c]], xv.at[c], sem))
            hs.append(pltpu.async_copy(py_hbm.at[idx_v.at[c]], yv.at[c], sem))
            hs.append(pltpu.async_copy(pz_hbm.at[idx_v.at[c]], zv.at[c], sem))
        for h in hs:
            h.wait()
        pltpu.sync_copy(xv, nx_o.at[wid])
        pltpu.sync_copy(yv, ny_o.at[wid])
        pltpu.sync_copy(zv, nz_o.at[wid])

    return k(nbr3, spx, spy, spz)


def _k6_edges(nx, ny, nz, sx, sy, sz, W1p, b1, W2, b2):
    def body(nx_ref, ny_ref, nz_ref, sx_ref, sy_ref, sz_ref,
             w1_ref, b1_ref, w2_ref, b2_ref, o_ref):
        dx = nx_ref[...] - sx_ref[...]
        dy = ny_ref[...] - sy_ref[...]
        dz = nz_ref[...] - sz_ref[...]
        norm = jnp.sqrt(dx * dx + dy * dy + dz * dz + EPS)
        x = dx / norm
        y = dy / norm
        z = dz / norm
        zero = jnp.zeros_like(x)
        fe = jnp.concatenate(
            [jnp.ones_like(x), x, y, z,
             x * y, y * z, 3.0 * z * z - 1.0, x * z, x * x - y * y,
             zero, zero, zero, zero, zero, zero, zero], axis=1)
        h1 = jnp.dot(fe, w1_ref[...], preferred_element_type=jnp.float32)
        h1 = jnp.maximum(h1 + b1_ref[...], 0.0)
        he = jnp.dot(h1, w2_ref[...], preferred_element_type=jnp.float32)
        o_ref[...] = he + b2_ref[...]

    f = pl.pallas_call(
        body, grid=(40,),
        in_specs=[pl.BlockSpec((1024, 1), lambda i: (i, 0))] * 6 +
                 [pl.BlockSpec((16, D), lambda i: (0, 0)),
                  pl.BlockSpec((1, D), lambda i: (0, 0)),
                  pl.BlockSpec((D, D), lambda i: (0, 0)),
                  pl.BlockSpec((1, D), lambda i: (0, 0))],
        out_specs=pl.BlockSpec((1024, D), lambda i: (i, 0)),
        out_shape=jax.ShapeDtypeStruct((K * KNN, D), jnp.float32))
    return f(nx, ny, nz, sx, sy, sz, W1p, b1, W2, b2)


def kernel(hn, pos, batch, w_score, W1, b1, W2, b2):
    score = _k1_score(hn, w_score)
    s1 = score.reshape(N)
    spad = jnp.concatenate(
        [s1, jnp.full((NP - N,), -jnp.inf, dtype=jnp.float32)])
    rank = _k2_rank(spad.reshape(NP, 1), spad.reshape(1, NP))
    rank_full = jnp.concatenate(
        [rank[:N, 0], N + jnp.arange(NS - N, dtype=jnp.int32)])
    ivals = jnp.arange(NS, dtype=jnp.int32)
    idx_by_rank = _k3_scatter(rank_full.reshape(32, 4, 80),
                              ivals.reshape(32, 4, 80))
    posT = pos.T
    ghn, spx, spy, spz, sbatch, ssc = _k4_gather(
        idx_by_rank, hn, posT[0], posT[1], posT[2], batch, s1)

    sub_hn, nbr = _k5_knn(
        ghn, ssc.reshape(KP, 1),
        spx.reshape(KP, 1), spy.reshape(KP, 1), spz.reshape(KP, 1),
        spx.reshape(1, KP), spy.reshape(1, KP), spz.reshape(1, KP))

    nxg, nyg, nzg = _k5b_coords(nbr.reshape(32, 10, 128), spx, spy, spz)

    W1p = jnp.pad(W1, ((0, 16 - W1.shape[0]), (0, 0)))
    he = _k6_edges(
        nxg.reshape(EP, 1), nyg.reshape(EP, 1), nzg.reshape(EP, 1),
        jnp.broadcast_to(spx[:, None], (KP, KNN)).reshape(EP, 1),
        jnp.broadcast_to(spy[:, None], (KP, KNN)).reshape(EP, 1),
        jnp.broadcast_to(spz[:, None], (KP, KNN)).reshape(EP, 1),
        W1p, b1.reshape(1, D), W2, b2.reshape(1, D))

    sub_pos = jnp.stack([spx[:K], spy[:K], spz[:K]], axis=1)
    sub_batch = sbatch[:K]
    return sub_hn, sub_pos, sub_batch, he
